# baseline (speedup 1.0000x reference)
import jax
import jax.numpy as jnp
from jax import lax
from jax.experimental import pallas as pl
from jax.experimental.pallas import tpu as pltpu
from jax.experimental.pallas import tpu_sc as plsc

N = 10000
D = 128
E = 320000
NC = 2
NS = 16
NW = NC * NS
CHUNK = 128
NCHUNKS = 2560
SC_C0 = 160
SC_C1 = 0
DEG_C0 = 80
DEG_C1 = 80
EPAD = NCHUNKS * CHUNK
NPAD = 10240
ROWS_PER_SUB = NPAD // NS
BR = 1024

_mesh = plsc.VectorSubcoreMesh(core_axis_name="c", subcore_axis_name="s")


def _split(c, s, c0_chunks, c1_chunks):
    my_chunks = jnp.where(c == 0, c0_chunks, c1_chunks)
    base = jnp.where(c == 0, s * c0_chunks, NS * c0_chunks + s * c1_chunks)
    return my_chunks, base


def _deg_body(idx_hbm, ones_hbm, zeros_hbm, out_hbm, ibuf, ones_v, acc,
              isem0, isem1):
    c = lax.axis_index("c")
    s = lax.axis_index("s")
    my_chunks, base = _split(c, s, DEG_C0, DEG_C1)
    pltpu.sync_copy(ones_hbm, ones_v)
    pltpu.sync_copy(idx_hbm.at[base], ibuf.at[0])
    pltpu.async_copy(idx_hbm.at[base + 1], ibuf.at[1], isem1)
    pltpu.sync_copy(zeros_hbm.at[pl.ds(s * ROWS_PER_SUB, ROWS_PER_SUB)],
                    acc.at[pl.ds(s * ROWS_PER_SUB, ROWS_PER_SUB)])
    plsc.subcore_barrier()

    def body(jj, carry):
        j0 = base + 2 * jj
        more = jj + 1 < my_chunks // 2
        pltpu.sync_copy(ones_v, acc.at[ibuf.at[0].at[1]], add=True)

        @pl.when(more)
        def _():
            pltpu.async_copy(idx_hbm.at[j0 + 2], ibuf.at[0], isem0)

        pltpu.make_async_copy(idx_hbm.at[j0 + 1], ibuf.at[1], isem1).wait()
        pltpu.sync_copy(ones_v, acc.at[ibuf.at[1].at[1]], add=True)

        @pl.when(more)
        def _():
            pltpu.make_async_copy(idx_hbm.at[j0 + 2], ibuf.at[0], isem0).wait()
            pltpu.async_copy(idx_hbm.at[j0 + 3], ibuf.at[1], isem1)

        return carry

    lax.fori_loop(0, my_chunks // 2, body, 0)
    plsc.subcore_barrier()
    pltpu.sync_copy(acc.at[pl.ds(s * ROWS_PER_SUB, ROWS_PER_SUB)],
                    out_hbm.at[c].at[pl.ds(s * ROWS_PER_SUB, ROWS_PER_SUB)])


_deg_call = pl.kernel(
    _deg_body,
    out_type=jax.ShapeDtypeStruct((NC, NPAD), jnp.float32),
    mesh=_mesh,
    scratch_types=[
        pltpu.VMEM((2, 2, CHUNK), jnp.int32),
        pltpu.VMEM((CHUNK,), jnp.float32),
        pltpu.VMEM_SHARED((NPAD,), jnp.float32),
        pltpu.SemaphoreType.DMA,
        pltpu.SemaphoreType.DMA,
    ],
)


def _scatter_body(u_hbm, idx_hbm, zeros_hbm, out_hbm,
                  ibuf, gbuf, acc, gsem0, gsem1, isem0, isem1):
    c = lax.axis_index("c")
    s = lax.axis_index("s")
    my_chunks, base = _split(c, s, SC_C0, SC_C1)
    @pl.when(my_chunks > 0)
    def _():
        pltpu.sync_copy(idx_hbm.at[base], ibuf.at[0])
        pltpu.async_copy(u_hbm.at[ibuf.at[0].at[0]], gbuf.at[0], gsem0)
        pltpu.async_copy(idx_hbm.at[base + 1], ibuf.at[1], isem1)
    for k in range(ROWS_PER_SUB // CHUNK):
        pltpu.sync_copy(zeros_hbm,
                        acc.at[pl.ds(s * ROWS_PER_SUB + k * CHUNK, CHUNK)])
    plsc.subcore_barrier()

    def body(jj, carry):
        j0 = base + 2 * jj
        more = jj + 1 < my_chunks // 2
        pltpu.make_async_copy(idx_hbm.at[j0 + 1], ibuf.at[1], isem1).wait()
        pltpu.make_async_copy(u_hbm.at[ibuf.at[0].at[0]], gbuf.at[0],
                              gsem0).wait()
        pltpu.async_copy(u_hbm.at[ibuf.at[1].at[0]], gbuf.at[1], gsem1)
        pltpu.sync_copy(gbuf.at[0], acc.at[ibuf.at[0].at[1]], add=True)

        @pl.when(more)
        def _():
            pltpu.async_copy(idx_hbm.at[j0 + 2], ibuf.at[0], isem0)

        pltpu.make_async_copy(u_hbm.at[ibuf.at[1].at[0]], gbuf.at[1],
                              gsem1).wait()

        @pl.when(more)
        def _():
            pltpu.make_async_copy(idx_hbm.at[j0 + 2], ibuf.at[0], isem0).wait()
            pltpu.async_copy(u_hbm.at[ibuf.at[0].at[0]], gbuf.at[0], gsem0)

        pltpu.sync_copy(gbuf.at[1], acc.at[ibuf.at[1].at[1]], add=True)

        @pl.when(more)
        def _():
            pltpu.async_copy(idx_hbm.at[j0 + 3], ibuf.at[1], isem1)

        return carry

    lax.fori_loop(0, my_chunks // 2, body, 0)
    plsc.subcore_barrier()
    pltpu.sync_copy(acc.at[pl.ds(s * ROWS_PER_SUB, ROWS_PER_SUB)],
                    out_hbm.at[c].at[pl.ds(s * ROWS_PER_SUB, ROWS_PER_SUB)])


_scatter_call = pl.kernel(
    _scatter_body,
    out_type=jax.ShapeDtypeStruct((NC, NPAD, D), jnp.float32),
    mesh=_mesh,
    scratch_types=[
        pltpu.VMEM((2, 2, CHUNK), jnp.int32),
        pltpu.VMEM((2, CHUNK, D), jnp.float32),
        pltpu.VMEM_SHARED((NPAD, D), jnp.float32),
        pltpu.SemaphoreType.DMA,
        pltpu.SemaphoreType.DMA,
        pltpu.SemaphoreType.DMA,
        pltpu.SemaphoreType.DMA,
    ],
)


def _tc1_body(degp_ref, x_ref, w_ref, u_ref, dinv_ref):
    dp = degp_ref[...]
    dinv = lax.rsqrt(dp[:, 0:1] + dp[:, 1:2] + 1.0)
    dinv_ref[...] = dinv
    u_ref[...] = jnp.dot(x_ref[...], w_ref[...],
                         preferred_element_type=jnp.float32) * dinv


def _tc2_body(v_ref, u1_ref, dinv_ref, b1_ref, w2_ref, h1_ref, u2_ref):
    v = v_ref[0] + v_ref[1] + u1_ref[...]
    dinv = dinv_ref[...]
    h1 = jnp.maximum(dinv * v + b1_ref[...], 0.0)
    h1_ref[...] = h1
    u2_ref[...] = jnp.dot(h1, w2_ref[...],
                          preferred_element_type=jnp.float32) * dinv


def _tc3_body(v_ref, u2_ref, dinv_ref, b2_ref, h1_ref, out_ref):
    v = v_ref[0] + v_ref[1] + u2_ref[...]
    out_ref[...] = (jnp.maximum(dinv_ref[...] * v + b2_ref[...], 0.0)
                    + h1_ref[...])


_GRID = (NPAD // BR,)
_row_spec = pl.BlockSpec((BR, D), lambda i: (i, 0))
_v_spec = pl.BlockSpec((NC, BR, D), lambda i: (0, i, 0))
_dinv_spec = pl.BlockSpec((BR, 1), lambda i: (i, 0))
_w_spec = pl.BlockSpec((D, D), lambda i: (0, 0))
_b_spec = pl.BlockSpec((1, D), lambda i: (0, 0))

_tc1 = pl.pallas_call(
    _tc1_body,
    grid=_GRID,
    in_specs=[pl.BlockSpec((BR, 2), lambda i: (i, 0)), _row_spec, _w_spec],
    out_specs=[_row_spec, _dinv_spec],
    out_shape=[jax.ShapeDtypeStruct((NPAD, D), jnp.float32),
               jax.ShapeDtypeStruct((NPAD, 1), jnp.float32)],
)

_tc2 = pl.pallas_call(
    _tc2_body,
    grid=_GRID,
    in_specs=[_v_spec, _row_spec, _dinv_spec, _b_spec, _w_spec],
    out_specs=[_row_spec, _row_spec],
    out_shape=[jax.ShapeDtypeStruct((NPAD, D), jnp.float32),
               jax.ShapeDtypeStruct((NPAD, D), jnp.float32)],
)

_tc3 = pl.pallas_call(
    _tc3_body,
    grid=_GRID,
    in_specs=[_v_spec, _row_spec, _dinv_spec, _b_spec, _row_spec],
    out_specs=_row_spec,
    out_shape=jax.ShapeDtypeStruct((NPAD, D), jnp.float32),
)


def kernel(x, edge_index, W1, b1, W2, b2):
    src = edge_index[0].astype(jnp.int32)
    dst = edge_index[1].astype(jnp.int32)
    pad = jnp.full((EPAD - E,), N, jnp.int32)
    src_r = jnp.concatenate([src, pad]).reshape(NCHUNKS, 1, CHUNK)
    dst_r = jnp.concatenate([dst, pad]).reshape(NCHUNKS, 1, CHUNK)
    idx_r = jnp.concatenate([src_r, dst_r], axis=1)
    x_pad = jnp.zeros((NPAD, D), jnp.float32).at[:N, :].set(x)
    ones_c = jnp.ones((CHUNK,), jnp.float32)
    zeros_nd = jnp.zeros((CHUNK, D), jnp.float32)
    zeros_n = jnp.zeros((NPAD,), jnp.float32)

    degp = _deg_call(idx_r, ones_c, zeros_n)
    u1, dinv = _tc1(degp.T, x_pad, W1)
    v1 = _scatter_call(u1, idx_r, zeros_nd)
    h1, u2 = _tc2(v1, u1, dinv, b1.reshape(1, D), W2)
    v2 = _scatter_call(u2, idx_r, zeros_nd)
    out = _tc3(v2, u2, dinv, b2.reshape(1, D), h1)
    return out[:N]


# speedup vs baseline: 1.1851x; 1.1851x over previous
import jax
import jax.numpy as jnp
from jax import lax
from jax.experimental import pallas as pl
from jax.experimental.pallas import tpu as pltpu
from jax.experimental.pallas import tpu_sc as plsc

N = 10000
D = 128
E = 320000
NC = 2
NS = 16
NW = NC * NS
CHUNK = 128
CHUNKS = 80
NCHUNKS = NW * CHUNKS
EPAD = NCHUNKS * CHUNK
NPAD = 10240
ROWS_PER_SUB = NPAD // NS
HALF = CHUNK // 2
BR = 1024

_mesh = plsc.VectorSubcoreMesh(core_axis_name="c", subcore_axis_name="s")


def _deg_body(idx_hbm, ones_hbm, zeros_hbm, out_hbm, ibuf, ones_v, acc,
              isem0, isem1):
    c = lax.axis_index("c")
    s = lax.axis_index("s")
    wid = s * NC + c
    base = wid * CHUNKS
    pltpu.sync_copy(ones_hbm, ones_v)
    pltpu.sync_copy(idx_hbm.at[base], ibuf.at[0])
    pltpu.async_copy(idx_hbm.at[base + 1], ibuf.at[1], isem1)
    pltpu.sync_copy(zeros_hbm.at[pl.ds(s * ROWS_PER_SUB, ROWS_PER_SUB)],
                    acc.at[pl.ds(s * ROWS_PER_SUB, ROWS_PER_SUB)])
    plsc.subcore_barrier()

    def body(jj, carry):
        j0 = base + 2 * jj
        more = jj + 1 < CHUNKS // 2
        pltpu.sync_copy(ones_v, acc.at[ibuf.at[0].at[1]], add=True)

        @pl.when(more)
        def _():
            pltpu.async_copy(idx_hbm.at[j0 + 2], ibuf.at[0], isem0)

        pltpu.make_async_copy(idx_hbm.at[j0 + 1], ibuf.at[1], isem1).wait()
        pltpu.sync_copy(ones_v, acc.at[ibuf.at[1].at[1]], add=True)

        @pl.when(more)
        def _():
            pltpu.make_async_copy(idx_hbm.at[j0 + 2], ibuf.at[0], isem0).wait()
            pltpu.async_copy(idx_hbm.at[j0 + 3], ibuf.at[1], isem1)

        return carry

    lax.fori_loop(0, CHUNKS // 2, body, 0)
    plsc.subcore_barrier()
    pltpu.sync_copy(acc.at[pl.ds(s * ROWS_PER_SUB, ROWS_PER_SUB)],
                    out_hbm.at[c].at[pl.ds(s * ROWS_PER_SUB, ROWS_PER_SUB)])


_deg_call = pl.kernel(
    _deg_body,
    out_type=jax.ShapeDtypeStruct((NC, NPAD), jnp.float32),
    mesh=_mesh,
    scratch_types=[
        pltpu.VMEM((2, 2, CHUNK), jnp.int32),
        pltpu.VMEM((CHUNK,), jnp.float32),
        pltpu.VMEM_SHARED((NPAD,), jnp.float32),
        pltpu.SemaphoreType.DMA,
        pltpu.SemaphoreType.DMA,
    ],
)


def _issue_gather(u_hbm, ibuf, gbuf, k, semL, semR):
    idx = ibuf.at[k].at[0]
    pltpu.async_copy(u_hbm.at[idx.at[pl.ds(0, HALF)]],
                     gbuf.at[k].at[pl.ds(0, HALF)], semL)
    pltpu.async_copy(u_hbm.at[idx.at[pl.ds(HALF, HALF)]],
                     gbuf.at[k].at[pl.ds(HALF, HALF)], semR)


def _wait_gather(u_hbm, ibuf, gbuf, k, semL, semR):
    idx = ibuf.at[k].at[0]
    pltpu.make_async_copy(u_hbm.at[idx.at[pl.ds(0, HALF)]],
                          gbuf.at[k].at[pl.ds(0, HALF)], semL).wait()
    pltpu.make_async_copy(u_hbm.at[idx.at[pl.ds(HALF, HALF)]],
                          gbuf.at[k].at[pl.ds(HALF, HALF)], semR).wait()


def _scatter_body(u_hbm, idx_hbm, zeros_hbm, out_hbm,
                  ibuf, gbuf, acc, g0L, g0R, g1L, g1R, isem0, isem1):
    c = lax.axis_index("c")
    s = lax.axis_index("s")
    wid = s * NC + c
    base = wid * CHUNKS
    row0 = s * ROWS_PER_SUB
    pltpu.sync_copy(idx_hbm.at[base], ibuf.at[0])
    _issue_gather(u_hbm, ibuf, gbuf, 0, g0L, g0R)
    pltpu.async_copy(idx_hbm.at[base + 1], ibuf.at[1], isem1)
    for k in range(ROWS_PER_SUB // CHUNK):
        pltpu.sync_copy(zeros_hbm, acc.at[pl.ds(row0 + k * CHUNK, CHUNK)])
    plsc.subcore_barrier()

    def body(jj, carry):
        j0 = base + 2 * jj
        more = jj + 1 < CHUNKS // 2
        pltpu.make_async_copy(idx_hbm.at[j0 + 1], ibuf.at[1], isem1).wait()
        _issue_gather(u_hbm, ibuf, gbuf, 1, g1L, g1R)
        _wait_gather(u_hbm, ibuf, gbuf, 0, g0L, g0R)
        pltpu.sync_copy(gbuf.at[0], acc.at[ibuf.at[0].at[1]], add=True)

        @pl.when(more)
        def _():
            pltpu.async_copy(idx_hbm.at[j0 + 2], ibuf.at[0], isem0)
            pltpu.make_async_copy(idx_hbm.at[j0 + 2], ibuf.at[0], isem0).wait()

        _wait_gather(u_hbm, ibuf, gbuf, 1, g1L, g1R)

        @pl.when(more)
        def _():
            _issue_gather(u_hbm, ibuf, gbuf, 0, g0L, g0R)
        pltpu.sync_copy(gbuf.at[1], acc.at[ibuf.at[1].at[1]], add=True)

        @pl.when(more)
        def _():
            pltpu.async_copy(idx_hbm.at[j0 + 3], ibuf.at[1], isem1)

        return carry

    lax.fori_loop(0, CHUNKS // 2, body, 0)
    plsc.subcore_barrier()
    pltpu.sync_copy(acc.at[pl.ds(row0, ROWS_PER_SUB)],
                    out_hbm.at[c].at[pl.ds(row0, ROWS_PER_SUB)])


_scatter_call = pl.kernel(
    _scatter_body,
    out_type=jax.ShapeDtypeStruct((NC, NPAD, D), jnp.float32),
    mesh=_mesh,
    scratch_types=[
        pltpu.VMEM((2, 2, CHUNK), jnp.int32),
        pltpu.VMEM((2, CHUNK, D), jnp.float32),
        pltpu.VMEM_SHARED((NPAD, D), jnp.float32),
        pltpu.SemaphoreType.DMA,
        pltpu.SemaphoreType.DMA,
        pltpu.SemaphoreType.DMA,
        pltpu.SemaphoreType.DMA,
        pltpu.SemaphoreType.DMA,
        pltpu.SemaphoreType.DMA,
    ],
)


def _tc1_body(degp_ref, x_ref, w_ref, u_ref, dinv_ref):
    dp = degp_ref[...]
    dinv = lax.rsqrt(dp[:, 0:1] + dp[:, 1:2] + 1.0)
    dinv_ref[...] = dinv
    u_ref[...] = jnp.dot(x_ref[...], w_ref[...],
                         preferred_element_type=jnp.float32) * dinv


def _tc2_body(v_ref, u1_ref, dinv_ref, b1_ref, w2_ref, h1_ref, u2_ref):
    v = v_ref[0] + v_ref[1] + u1_ref[...]
    dinv = dinv_ref[...]
    h1 = jnp.maximum(dinv * v + b1_ref[...], 0.0)
    h1_ref[...] = h1
    u2_ref[...] = jnp.dot(h1, w2_ref[...],
                          preferred_element_type=jnp.float32) * dinv


def _tc3_body(v_ref, u2_ref, dinv_ref, b2_ref, h1_ref, out_ref):
    v = v_ref[0] + v_ref[1] + u2_ref[...]
    out_ref[...] = (jnp.maximum(dinv_ref[...] * v + b2_ref[...], 0.0)
                    + h1_ref[...])


_GRID = (NPAD // BR,)
_row_spec = pl.BlockSpec((BR, D), lambda i: (i, 0))
_v_spec = pl.BlockSpec((NC, BR, D), lambda i: (0, i, 0))
_dinv_spec = pl.BlockSpec((BR, 1), lambda i: (i, 0))
_w_spec = pl.BlockSpec((D, D), lambda i: (0, 0))
_b_spec = pl.BlockSpec((1, D), lambda i: (0, 0))

_tc1 = pl.pallas_call(
    _tc1_body,
    grid=_GRID,
    in_specs=[pl.BlockSpec((BR, 2), lambda i: (i, 0)), _row_spec, _w_spec],
    out_specs=[_row_spec, _dinv_spec],
    out_shape=[jax.ShapeDtypeStruct((NPAD, D), jnp.float32),
               jax.ShapeDtypeStruct((NPAD, 1), jnp.float32)],
)

_tc2 = pl.pallas_call(
    _tc2_body,
    grid=_GRID,
    in_specs=[_v_spec, _row_spec, _dinv_spec, _b_spec, _w_spec],
    out_specs=[_row_spec, _row_spec],
    out_shape=[jax.ShapeDtypeStruct((NPAD, D), jnp.float32),
               jax.ShapeDtypeStruct((NPAD, D), jnp.float32)],
)

_tc3 = pl.pallas_call(
    _tc3_body,
    grid=_GRID,
    in_specs=[_v_spec, _row_spec, _dinv_spec, _b_spec, _row_spec],
    out_specs=_row_spec,
    out_shape=jax.ShapeDtypeStruct((NPAD, D), jnp.float32),
)


def kernel(x, edge_index, W1, b1, W2, b2):
    src = edge_index[0].astype(jnp.int32)
    dst = edge_index[1].astype(jnp.int32)
    pad = jnp.full((EPAD - E,), N, jnp.int32)
    src_r = jnp.concatenate([src, pad]).reshape(NCHUNKS, 1, CHUNK)
    dst_r = jnp.concatenate([dst, pad]).reshape(NCHUNKS, 1, CHUNK)
    idx_r = jnp.concatenate([src_r, dst_r], axis=1)
    x_pad = jnp.zeros((NPAD, D), jnp.float32).at[:N, :].set(x)
    ones_c = jnp.ones((CHUNK,), jnp.float32)
    zeros_nd = jnp.zeros((CHUNK, D), jnp.float32)
    zeros_n = jnp.zeros((NPAD,), jnp.float32)

    degp = _deg_call(idx_r, ones_c, zeros_n)
    u1, dinv = _tc1(degp.T, x_pad, W1)
    v1 = _scatter_call(u1, idx_r, zeros_nd)
    h1, u2 = _tc2(v1, u1, dinv, b1.reshape(1, D), W2)
    v2 = _scatter_call(u2, idx_r, zeros_nd)
    out = _tc3(v2, u2, dinv, b2.reshape(1, D), h1)
    return out[:N]


# speedup vs baseline: 1.2930x; 1.0910x over previous
import jax
import jax.numpy as jnp
from jax import lax
from jax.experimental import pallas as pl
from jax.experimental.pallas import tpu as pltpu
from jax.experimental.pallas import tpu_sc as plsc

N = 10000
D = 128
E = 320000
NC = 2
NS = 16
NW = NC * NS
CHUNK = 128
CHUNKS = 80
NCHUNKS = NW * CHUNKS
EPAD = NCHUNKS * CHUNK
NPAD = 10240
ROWS_PER_SUB = NPAD // NS
HALF = CHUNK // 2
SC_C0 = 124
SC_C1 = 36
BR = 1024

_mesh = plsc.VectorSubcoreMesh(core_axis_name="c", subcore_axis_name="s")


def _deg_body(idx_hbm, ones_hbm, zeros_hbm, out_hbm, ibuf, ones_v, acc,
              isem0, isem1):
    c = lax.axis_index("c")
    s = lax.axis_index("s")
    wid = s * NC + c
    base = wid * CHUNKS
    pltpu.sync_copy(ones_hbm, ones_v)
    pltpu.sync_copy(idx_hbm.at[base], ibuf.at[0])
    pltpu.async_copy(idx_hbm.at[base + 1], ibuf.at[1], isem1)
    pltpu.sync_copy(zeros_hbm.at[pl.ds(s * ROWS_PER_SUB, ROWS_PER_SUB)],
                    acc.at[pl.ds(s * ROWS_PER_SUB, ROWS_PER_SUB)])
    plsc.subcore_barrier()

    def body(jj, carry):
        j0 = base + 2 * jj
        more = jj + 1 < CHUNKS // 2
        pltpu.sync_copy(ones_v, acc.at[ibuf.at[0].at[1]], add=True)

        @pl.when(more)
        def _():
            pltpu.async_copy(idx_hbm.at[j0 + 2], ibuf.at[0], isem0)

        pltpu.make_async_copy(idx_hbm.at[j0 + 1], ibuf.at[1], isem1).wait()
        pltpu.sync_copy(ones_v, acc.at[ibuf.at[1].at[1]], add=True)

        @pl.when(more)
        def _():
            pltpu.make_async_copy(idx_hbm.at[j0 + 2], ibuf.at[0], isem0).wait()
            pltpu.async_copy(idx_hbm.at[j0 + 3], ibuf.at[1], isem1)

        return carry

    lax.fori_loop(0, CHUNKS // 2, body, 0)
    plsc.subcore_barrier()
    pltpu.sync_copy(acc.at[pl.ds(s * ROWS_PER_SUB, ROWS_PER_SUB)],
                    out_hbm.at[c].at[pl.ds(s * ROWS_PER_SUB, ROWS_PER_SUB)])


_deg_call = pl.kernel(
    _deg_body,
    out_type=jax.ShapeDtypeStruct((NC, NPAD), jnp.float32),
    mesh=_mesh,
    scratch_types=[
        pltpu.VMEM((2, 2, CHUNK), jnp.int32),
        pltpu.VMEM((CHUNK,), jnp.float32),
        pltpu.VMEM_SHARED((NPAD,), jnp.float32),
        pltpu.SemaphoreType.DMA,
        pltpu.SemaphoreType.DMA,
    ],
)


def _issue_gather(u_hbm, ibuf, gbuf, k, semL, semR):
    idx = ibuf.at[k].at[0]
    pltpu.async_copy(u_hbm.at[idx.at[pl.ds(0, HALF)]],
                     gbuf.at[k].at[pl.ds(0, HALF)], semL)
    pltpu.async_copy(u_hbm.at[idx.at[pl.ds(HALF, HALF)]],
                     gbuf.at[k].at[pl.ds(HALF, HALF)], semR)


def _wait_gather(u_hbm, ibuf, gbuf, k, semL, semR):
    idx = ibuf.at[k].at[0]
    pltpu.make_async_copy(u_hbm.at[idx.at[pl.ds(0, HALF)]],
                          gbuf.at[k].at[pl.ds(0, HALF)], semL).wait()
    pltpu.make_async_copy(u_hbm.at[idx.at[pl.ds(HALF, HALF)]],
                          gbuf.at[k].at[pl.ds(HALF, HALF)], semR).wait()


def _scatter_body(u_hbm, idx_hbm, zeros_hbm, out_hbm,
                  ibuf, gbuf, acc, g0L, g0R, g1L, g1R, isem0, isem1):
    c = lax.axis_index("c")
    s = lax.axis_index("s")
    my_chunks = jnp.where(c == 0, SC_C0, SC_C1)
    base = jnp.where(c == 0, s * SC_C0, NS * SC_C0 + s * SC_C1)
    row0 = s * ROWS_PER_SUB
    pltpu.sync_copy(idx_hbm.at[base], ibuf.at[0])
    _issue_gather(u_hbm, ibuf, gbuf, 0, g0L, g0R)
    pltpu.async_copy(idx_hbm.at[base + 1], ibuf.at[1], isem1)
    for k in range(ROWS_PER_SUB // CHUNK):
        pltpu.sync_copy(zeros_hbm, acc.at[pl.ds(row0 + k * CHUNK, CHUNK)])
    plsc.subcore_barrier()

    def body(jj, carry):
        j0 = base + 2 * jj
        more = jj + 1 < my_chunks // 2
        pltpu.make_async_copy(idx_hbm.at[j0 + 1], ibuf.at[1], isem1).wait()
        _issue_gather(u_hbm, ibuf, gbuf, 1, g1L, g1R)
        _wait_gather(u_hbm, ibuf, gbuf, 0, g0L, g0R)
        pltpu.sync_copy(gbuf.at[0], acc.at[ibuf.at[0].at[1]], add=True)

        @pl.when(more)
        def _():
            pltpu.async_copy(idx_hbm.at[j0 + 2], ibuf.at[0], isem0)
            pltpu.make_async_copy(idx_hbm.at[j0 + 2], ibuf.at[0], isem0).wait()

        _wait_gather(u_hbm, ibuf, gbuf, 1, g1L, g1R)

        @pl.when(more)
        def _():
            _issue_gather(u_hbm, ibuf, gbuf, 0, g0L, g0R)
        pltpu.sync_copy(gbuf.at[1], acc.at[ibuf.at[1].at[1]], add=True)

        @pl.when(more)
        def _():
            pltpu.async_copy(idx_hbm.at[j0 + 3], ibuf.at[1], isem1)

        return carry

    lax.fori_loop(0, my_chunks // 2, body, 0)
    plsc.subcore_barrier()
    pltpu.sync_copy(acc.at[pl.ds(row0, ROWS_PER_SUB)],
                    out_hbm.at[c].at[pl.ds(row0, ROWS_PER_SUB)])


_scatter_call = pl.kernel(
    _scatter_body,
    out_type=jax.ShapeDtypeStruct((NC, NPAD, D), jnp.float32),
    mesh=_mesh,
    scratch_types=[
        pltpu.VMEM((2, 2, CHUNK), jnp.int32),
        pltpu.VMEM((2, CHUNK, D), jnp.float32),
        pltpu.VMEM_SHARED((NPAD, D), jnp.float32),
        pltpu.SemaphoreType.DMA,
        pltpu.SemaphoreType.DMA,
        pltpu.SemaphoreType.DMA,
        pltpu.SemaphoreType.DMA,
        pltpu.SemaphoreType.DMA,
        pltpu.SemaphoreType.DMA,
    ],
)


def _tc1_body(degp_ref, x_ref, w_ref, u_ref, dinv_ref):
    dp = degp_ref[...]
    dinv = lax.rsqrt(dp[:, 0:1] + dp[:, 1:2] + 1.0)
    dinv_ref[...] = dinv
    u_ref[...] = jnp.dot(x_ref[...], w_ref[...],
                         preferred_element_type=jnp.float32) * dinv


def _tc2_body(v_ref, u1_ref, dinv_ref, b1_ref, w2_ref, h1_ref, u2_ref):
    v = v_ref[0] + v_ref[1] + u1_ref[...]
    dinv = dinv_ref[...]
    h1 = jnp.maximum(dinv * v + b1_ref[...], 0.0)
    h1_ref[...] = h1
    u2_ref[...] = jnp.dot(h1, w2_ref[...],
                          preferred_element_type=jnp.float32) * dinv


def _tc3_body(v_ref, u2_ref, dinv_ref, b2_ref, h1_ref, out_ref):
    v = v_ref[0] + v_ref[1] + u2_ref[...]
    out_ref[...] = (jnp.maximum(dinv_ref[...] * v + b2_ref[...], 0.0)
                    + h1_ref[...])


_GRID = (NPAD // BR,)
_row_spec = pl.BlockSpec((BR, D), lambda i: (i, 0))
_v_spec = pl.BlockSpec((NC, BR, D), lambda i: (0, i, 0))
_dinv_spec = pl.BlockSpec((BR, 1), lambda i: (i, 0))
_w_spec = pl.BlockSpec((D, D), lambda i: (0, 0))
_b_spec = pl.BlockSpec((1, D), lambda i: (0, 0))

_tc1 = pl.pallas_call(
    _tc1_body,
    grid=_GRID,
    in_specs=[pl.BlockSpec((BR, 2), lambda i: (i, 0)), _row_spec, _w_spec],
    out_specs=[_row_spec, _dinv_spec],
    out_shape=[jax.ShapeDtypeStruct((NPAD, D), jnp.float32),
               jax.ShapeDtypeStruct((NPAD, 1), jnp.float32)],
)

_tc2 = pl.pallas_call(
    _tc2_body,
    grid=_GRID,
    in_specs=[_v_spec, _row_spec, _dinv_spec, _b_spec, _w_spec],
    out_specs=[_row_spec, _row_spec],
    out_shape=[jax.ShapeDtypeStruct((NPAD, D), jnp.float32),
               jax.ShapeDtypeStruct((NPAD, D), jnp.float32)],
)

_tc3 = pl.pallas_call(
    _tc3_body,
    grid=_GRID,
    in_specs=[_v_spec, _row_spec, _dinv_spec, _b_spec, _row_spec],
    out_specs=_row_spec,
    out_shape=jax.ShapeDtypeStruct((NPAD, D), jnp.float32),
)


def kernel(x, edge_index, W1, b1, W2, b2):
    src = edge_index[0].astype(jnp.int32)
    dst = edge_index[1].astype(jnp.int32)
    pad = jnp.full((EPAD - E,), N, jnp.int32)
    src_r = jnp.concatenate([src, pad]).reshape(NCHUNKS, 1, CHUNK)
    dst_r = jnp.concatenate([dst, pad]).reshape(NCHUNKS, 1, CHUNK)
    idx_r = jnp.concatenate([src_r, dst_r], axis=1)
    x_pad = jnp.zeros((NPAD, D), jnp.float32).at[:N, :].set(x)
    ones_c = jnp.ones((CHUNK,), jnp.float32)
    zeros_nd = jnp.zeros((CHUNK, D), jnp.float32)
    zeros_n = jnp.zeros((NPAD,), jnp.float32)

    degp = _deg_call(idx_r, ones_c, zeros_n)
    u1, dinv = _tc1(degp.T, x_pad, W1)
    v1 = _scatter_call(u1, idx_r, zeros_nd)
    h1, u2 = _tc2(v1, u1, dinv, b1.reshape(1, D), W2)
    v2 = _scatter_call(u2, idx_r, zeros_nd)
    out = _tc3(v2, u2, dinv, b2.reshape(1, D), h1)
    return out[:N]
